# baseline (speedup 1.0000x reference)
import functools

import jax
import jax.numpy as jnp
from jax import lax
from jax.experimental import pallas as pl
from jax.experimental.pallas import tpu as pltpu
from jax.experimental.pallas import tpu_sc as plsc

B = 32
D = 2048
NLANES = 16
NCHUNK = D // NLANES

SC_ROWS = 8192
NW = 32
ROWS_PER_TILE = SC_ROWS // NW
STAGE = 16
IB = 4
JB = 8
N_STAGES = ROWS_PER_TILE // STAGE
N_BATCH = STAGE // IB

TC_BK = 2048
SC_BLOCKS = SC_ROWS // TC_BK


def _tc_body(q_ref, k_ref, score_ref, idx_ref, qn_ref, *,
             block_k: int, row0_blocks: int, total_k: int):
    i = pl.program_id(0)
    q = q_ref[...]
    k = k_ref[...]

    dots = lax.dot_general(
        q, k, (((1,), (1,)), ((), ())),
        preferred_element_type=jnp.float32,
    )

    q_norm = jnp.sqrt(jnp.sum(q * q, axis=1, keepdims=True))
    k_norm = jnp.sqrt(jnp.sum(k * k, axis=1, keepdims=True))
    cos = dots / (q_norm * k_norm.reshape(1, -1))

    gid = (row0_blocks + i) * block_k + jax.lax.broadcasted_iota(
        jnp.int32, cos.shape, 1)
    cos = jnp.where(gid < total_k, cos, -jnp.inf)

    blk_max = jnp.max(cos, axis=1, keepdims=True)
    blk_arg = jnp.min(
        jnp.where(cos == blk_max, gid, jnp.int32(2**31 - 1)),
        axis=1, keepdims=True,
    )

    @pl.when(i == 0)
    def _():
        score_ref[...] = blk_max
        idx_ref[...] = blk_arg
        qn_ref[...] = q_norm

    @pl.when(i > 0)
    def _():
        prev = score_ref[...]
        better = blk_max > prev
        score_ref[...] = jnp.where(better, blk_max, prev)
        idx_ref[...] = jnp.where(better, blk_arg, idx_ref[...])


def _tc_retrieve(queries, keys, row0: int, total_k: int):
    nb = pl.cdiv(total_k - row0, TC_BK)
    row0_blocks = row0 // TC_BK
    score, idx, qn = pl.pallas_call(
        functools.partial(_tc_body, block_k=TC_BK, row0_blocks=row0_blocks,
                          total_k=total_k),
        grid=(nb,),
        in_specs=[
            pl.BlockSpec((B, D), lambda i: (0, 0)),
            pl.BlockSpec((TC_BK, D),
                         lambda i, r0=row0_blocks: (r0 + i, 0)),
        ],
        out_specs=[
            pl.BlockSpec((B, 1), lambda i: (0, 0)),
            pl.BlockSpec((B, 1), lambda i: (0, 0)),
            pl.BlockSpec((B, 1), lambda i: (0, 0)),
        ],
        out_shape=[
            jax.ShapeDtypeStruct((B, 1), jnp.float32),
            jax.ShapeDtypeStruct((B, 1), jnp.int32),
            jax.ShapeDtypeStruct((B, 1), jnp.float32),
        ],
        compiler_params=pltpu.CompilerParams(
            dimension_semantics=("arbitrary",),
        ),
    )(queries, keys)
    return score.reshape(B), idx.reshape(B), qn.reshape(B)


def _sc_body(q_hbm, k_hbm, t_hbm, i_hbm, q_v, k_v, best_t_v, best_i_v):
    cid = lax.axis_index("c")
    sid = lax.axis_index("s")
    wid = cid * 16 + sid
    base = wid * ROWS_PER_TILE

    pltpu.sync_copy(q_hbm, q_v)

    lane = lax.broadcasted_iota(jnp.int32, (NLANES,), 0)
    neg_inf = jnp.full((NLANES,), -jnp.inf, jnp.float32)
    zeros_i = jnp.zeros((NLANES,), jnp.int32)

    def hsum(x):
        for dd in (8, 4, 2, 1):
            x = x + x.at[lane ^ dd].get(mode="promise_in_bounds")
        return x

    def stage_body(st, best):
        row0 = base + st * STAGE
        pltpu.sync_copy(k_hbm.at[pl.ds(row0, STAGE), :], k_v)

        def batch_body(bb, best):
            kb = bb * IB
            bts = list(best[:2])
            bis = list(best[2:])
            ksq_s = [None] * IB

            for jg in range(B // JB):
                with_ksq = jg == 0

                def chunk_body(c, carry):
                    accs = carry[:JB * IB]
                    ksqa = carry[JB * IB:]
                    kc = [k_v[kb + i, pl.ds(c * NLANES, NLANES)]
                          for i in range(IB)]
                    qc = [q_v[jg * JB + j, pl.ds(c * NLANES, NLANES)]
                          for j in range(JB)]
                    new_accs = tuple(
                        accs[j * IB + i] + qc[j] * kc[i]
                        for j in range(JB) for i in range(IB))
                    new_ksq = tuple(
                        ksqa[i] + kc[i] * kc[i]
                        for i in range(len(ksqa)))
                    return new_accs + new_ksq

                z = jnp.zeros((NLANES,), jnp.float32)
                init = (z,) * (JB * IB + (IB if with_ksq else 0))
                out = lax.fori_loop(0, NCHUNK, chunk_body, init)
                accs = out[:JB * IB]
                if with_ksq:
                    for i in range(IB):
                        ksq_s[i] = hsum(out[JB * IB + i])

                for j in range(JB):
                    jq = jg * JB + j
                    h, l = divmod(jq, NLANES)
                    lane_eq = lane == l
                    for i in range(IB):
                        dot = hsum(accs[j * IB + i])
                        t = dot * jnp.abs(dot) / ksq_s[i]
                        gidx = jnp.full((NLANES,), row0 + kb + i, jnp.int32)
                        upd = jnp.logical_and(lane_eq, t > bts[h])
                        bts[h] = jnp.where(upd, t, bts[h])
                        bis[h] = jnp.where(upd, gidx, bis[h])
            return (bts[0], bts[1], bis[0], bis[1])

        return lax.fori_loop(0, N_BATCH, batch_body, best)

    best = lax.fori_loop(0, N_STAGES, stage_body,
                         (neg_inf, neg_inf, zeros_i, zeros_i))

    best_t_v[pl.ds(0, NLANES)] = best[0]
    best_t_v[pl.ds(NLANES, NLANES)] = best[1]
    best_i_v[pl.ds(0, NLANES)] = best[2]
    best_i_v[pl.ds(NLANES, NLANES)] = best[3]
    pltpu.sync_copy(best_t_v, t_hbm.at[wid])
    pltpu.sync_copy(best_i_v, i_hbm.at[wid])


def _sc_retrieve(queries, keys):
    mesh = plsc.VectorSubcoreMesh(core_axis_name="c", subcore_axis_name="s")
    kfn = pl.kernel(
        _sc_body,
        out_type=[
            jax.ShapeDtypeStruct((NW, B), jnp.float32),
            jax.ShapeDtypeStruct((NW, B), jnp.int32),
        ],
        mesh=mesh,
        scratch_types=[
            pltpu.VMEM((B, D), jnp.float32),
            pltpu.VMEM((STAGE, D), jnp.float32),
            pltpu.VMEM((B,), jnp.float32),
            pltpu.VMEM((B,), jnp.int32),
        ],
    )
    return kfn(queries, keys)


def kernel(queries, keys):
    total_k = keys.shape[0]

    sc_t, sc_i = _sc_retrieve(queries, keys)
    tc_score, tc_idx, qn = _tc_retrieve(queries, keys, SC_ROWS, total_k)

    tile = jnp.argmax(sc_t, axis=0)
    cols = jnp.arange(B)
    best_t = sc_t[tile, cols]
    sc_idx = sc_i[tile, cols]
    sc_score = jnp.sign(best_t) * jnp.sqrt(jnp.abs(best_t)) / qn

    use_tc = tc_score > sc_score
    score = jnp.where(use_tc, tc_score, sc_score)
    idx = jnp.where(use_tc, tc_idx, sc_idx)
    return score, idx
